# Initial kernel scaffold; baseline (speedup 1.0000x reference)
import jax
import jax.numpy as jnp
from jax.experimental import pallas as pl


def kernel(unmasked_data, unmasked_idx, masked_idx, W_Gender, W_Department, W_Grade, W_Extracurricular_Activities, W_Internet_Access_at_Home, W_Parent_Education_Level, W_Family_Income_Level, W_num, W_pos):
    raise NotImplementedError("write your pallas kernel here")


# speedup vs baseline: 5.6270x; 5.6270x over previous
import jax
import jax.numpy as jnp
from jax.experimental import pallas as pl
from jax.experimental.pallas import tpu as pltpu

_FEAT = 8
_POS_DIM = 248
_ROW = _FEAT + _POS_DIM
_MAX_ROWS = 6
_NTAB = 7
_BLK = 512


def _encode_body(aid_ref, mid_ref, len_ref,
                 data_ref, tab_ref, wnum_ref, wpos_ref,
                 out_un_ref, out_m_ref):
    blk = data_ref.shape[0]
    n_un = data_ref.shape[1]
    n_m = out_m_ref.shape[1]
    n_pos = wpos_ref.shape[0]

    for c in range(n_un):
        aid = aid_ref[c]
        bid = jnp.minimum(aid, _NTAB)
        tid = jnp.minimum(bid, _NTAB - 1)
        pos_id = jnp.clip(aid, 0, n_pos - 1)
        pos_row = wpos_ref[pl.ds(pos_id, 1), :]
        tbl = tab_ref[pl.ds(tid, 1)]
        nrow = len_ref[tid]
        dcol = data_ref[:, c:c + 1]
        didx = jnp.clip(dcol.astype(jnp.int32), 0, nrow - 1)
        cat8 = jnp.zeros((blk, _FEAT), jnp.float32)
        for k in range(_MAX_ROWS):
            cat8 = cat8 + jnp.where(didx == k, 1.0, 0.0) * tbl[0, k:k + 1, :]
        num8 = dcol * wnum_ref[0:1, :]
        emb8 = jnp.where(bid == _NTAB, num8, cat8)
        row = jnp.concatenate(
            [emb8, jnp.broadcast_to(pos_row, (blk, _POS_DIM))], axis=1)
        out_un_ref[:, c, :] = row

    for c in range(n_m):
        mid = mid_ref[c]
        bid = jnp.minimum(mid, _NTAB - 1)
        tbl = tab_ref[pl.ds(bid, 1)]
        mrow = len_ref[bid] - 1
        vec8 = jnp.zeros((1, _FEAT), jnp.float32)
        for k in range(_MAX_ROWS):
            vec8 = vec8 + jnp.where(mrow == k, 1.0, 0.0) * tbl[0, k:k + 1, :]
        pos_id = jnp.clip(mid, 0, n_pos - 1)
        pos_row = wpos_ref[pl.ds(pos_id, 1), :]
        row = jnp.concatenate([vec8, pos_row], axis=1)
        out_m_ref[:, c, :] = jnp.broadcast_to(row, (blk, _ROW))


def kernel(unmasked_data, unmasked_idx, masked_idx, W_Gender, W_Department,
           W_Grade, W_Extracurricular_Activities, W_Internet_Access_at_Home,
           W_Parent_Education_Level, W_Family_Income_Level, W_num, W_pos):
    tables = [W_Gender, W_Department, W_Grade, W_Extracurricular_Activities,
              W_Internet_Access_at_Home, W_Parent_Education_Level,
              W_Family_Income_Level]
    bsz, n_un = unmasked_data.shape
    n_m = masked_idx.shape[1]
    stacked = jnp.stack(
        [jnp.pad(t, ((0, _MAX_ROWS - t.shape[0]), (0, 0))) for t in tables])
    lens = jnp.array([t.shape[0] for t in tables], jnp.int32)
    aid = unmasked_idx[0, :]
    mid = masked_idx[0, :]

    grid = (bsz // _BLK,)
    out_shapes = (
        jax.ShapeDtypeStruct((bsz, n_un, _ROW), jnp.float32),
        jax.ShapeDtypeStruct((bsz, n_m, _ROW), jnp.float32),
    )
    out_un, out_m = pl.pallas_call(
        _encode_body,
        grid=grid,
        in_specs=[
            pl.BlockSpec(memory_space=pltpu.SMEM),
            pl.BlockSpec(memory_space=pltpu.SMEM),
            pl.BlockSpec(memory_space=pltpu.SMEM),
            pl.BlockSpec((_BLK, n_un), lambda i: (i, 0)),
            pl.BlockSpec((_NTAB, _MAX_ROWS, _FEAT), lambda i: (0, 0, 0)),
            pl.BlockSpec((1, _FEAT), lambda i: (0, 0)),
            pl.BlockSpec(W_pos.shape, lambda i: (0, 0)),
        ],
        out_specs=[
            pl.BlockSpec((_BLK, n_un, _ROW), lambda i: (i, 0, 0)),
            pl.BlockSpec((_BLK, n_m, _ROW), lambda i: (i, 0, 0)),
        ],
        out_shape=out_shapes,
        compiler_params=pltpu.CompilerParams(
            dimension_semantics=("parallel",)),
    )(aid, mid, lens, unmasked_data, stacked, W_num, W_pos)
    return out_un, out_m
